# baseline (speedup 1.0000x reference)
import functools

import jax
import jax.numpy as jnp
from jax import lax
from jax.experimental import pallas as pl
from jax.experimental.pallas import tpu as pltpu
from jax.experimental.pallas import tpu_sc as plsc

H = 224
W = 224
C = 192
CP = 256
N = 1000
NP = 1024
NW = 32
BPW = NP // NW
OH = 7
OW = 7
NCH = C // 16
ROWS = OW * 16


def _axis_params(v):
    t = v.astype(jnp.int32)
    fl = jnp.where(t.astype(jnp.float32) > v, t - 1, t)
    lerp = v - fl.astype(jnp.float32)
    valid = jnp.where((v >= 0.0) & (v <= 223.0), 1.0, 0.0).astype(jnp.float32)
    lo = jnp.clip(fl, 0, 223)
    hi = jnp.clip(fl + 1, 0, 223)
    wlo = valid * (1.0 - lerp)
    whi = valid * lerp
    return lo, hi, wlo, whi


def _roialign_body(img, boxes, bidx, out,
                   boxsm, bism, idxa, idxb, wbuf, rowsa, rowsb, outrow,
                   sema, semb):
    c = lax.axis_index("c")
    s = lax.axis_index("s")
    wid = s * 2 + c
    lo = wid * BPW
    nb = jnp.minimum(BPW, jnp.maximum(0, N - lo))
    pltpu.sync_copy(boxes.at[pl.ds(lo * 4, BPW * 4)], boxsm.at[pl.ds(0, BPW * 4)])
    pltpu.sync_copy(bidx.at[pl.ds(lo, BPW)], bism.at[pl.ds(0, BPW)])

    lane = lax.iota(jnp.int32, 16)
    sy_hi = ((lane >> 3) & 1) == 1
    sx_hi = ((lane >> 2) & 1) == 1
    cy_hi = ((lane >> 1) & 1) == 1
    cx_hi = (lane & 1) == 1

    def box_loop(j, carry):
        bvec = boxsm[pl.ds(j * 4, 16)]
        bcy = bvec[0]
        bcx = bvec[1]
        bh = bvec[2]
        bw = bvec[3]
        y1 = bcy - bh * 0.5
        y2 = bcy + bh * 0.5
        x1 = bcx - bw * 0.5
        x2 = bcx + bw * 0.5
        bin_h = (y2 - y1) * (1.0 / 7.0)
        bin_w = (x2 - x1) * (1.0 / 7.0)
        gy1 = y1 + 0.25 * bin_h
        gy2 = y2 - 0.25 * bin_h
        gx1 = x1 + 0.25 * bin_w
        gx2 = x2 - 0.25 * bin_w
        hs = (gy2 - gy1) * ((H - 1.0) / 13.0)
        ws = (gx2 - gx1) * ((W - 1.0) / 13.0)
        y0f = gy1 * (H - 1.0)
        x0f = gx1 * (W - 1.0)
        base = bism[pl.ds(j, 16)][0] * (H * W)

        xpar = [_axis_params(x0f + float(ix) * ws) for ix in range(2 * OW)]

        def oy_loop(oy, carry2):
            oyf = (2 * oy).astype(jnp.float32)
            t0, b0, wt0, wb0 = _axis_params(y0f + oyf * hs)
            t1, b1, wt1, wb1 = _axis_params(y0f + (oyf + 1.0) * hs)
            y16 = jnp.where(cy_hi,
                            jnp.where(sy_hi, b1, b0),
                            jnp.where(sy_hi, t1, t0))
            wy16 = 0.25 * jnp.where(cy_hi,
                                    jnp.where(sy_hi, wb1, wb0),
                                    jnp.where(sy_hi, wt1, wt0))
            for ox in range(OW):
                l0, r0, wl0, wr0 = xpar[2 * ox]
                l1, r1, wl1, wr1 = xpar[2 * ox + 1]
                x16 = jnp.where(cx_hi,
                                jnp.where(sx_hi, r1, r0),
                                jnp.where(sx_hi, l1, l0))
                wx16 = jnp.where(cx_hi,
                                 jnp.where(sx_hi, wr1, wr0),
                                 jnp.where(sx_hi, wl1, wl0))
                p16 = (base + y16 * W + x16) * 2
                idxa[pl.ds(ox * 16, 16)] = p16
                idxb[pl.ds(ox * 16, 16)] = p16 + 1
                wbuf[pl.ds(ox * 16, 16)] = wy16 * wx16
            cpa = pltpu.async_copy(img.at[idxa], rowsa, sema)
            cpb = pltpu.async_copy(img.at[idxb], rowsb, semb)
            cpa.wait()
            cpb.wait()

            def bin_loop(ox, carry3):
                rbase = ox * 16
                wvec = wbuf[pl.ds(rbase, 16)]
                accs = [jnp.zeros((16,), jnp.float32) for _ in range(NCH)]
                for k in range(16):
                    wk = wvec[k]
                    for ch in range(8):
                        accs[ch] = (accs[ch]
                                    + rowsa[rbase + k, pl.ds(ch * 16, 16)] * wk)
                    for ch in range(8, NCH):
                        accs[ch] = (accs[ch]
                                    + rowsb[rbase + k, pl.ds((ch - 8) * 16, 16)] * wk)
                for ch in range(NCH):
                    outrow[ox, pl.ds(ch * 16, 16)] = accs[ch]
                return carry3

            lax.fori_loop(0, OW, bin_loop, None)
            pltpu.sync_copy(outrow, out.at[lo + j, oy])
            return carry2

        lax.fori_loop(0, OH, oy_loop, None)
        return carry

    lax.fori_loop(0, nb, box_loop, None)


_roialign_sc = functools.partial(
    pl.kernel,
    out_type=jax.ShapeDtypeStruct((N, OH, OW, C), jnp.float32),
    mesh=plsc.VectorSubcoreMesh(core_axis_name="c", subcore_axis_name="s"),
    scratch_types=[
        pltpu.VMEM((BPW * 4 + 16,), jnp.float32),
        pltpu.VMEM((BPW + 16,), jnp.int32),
        pltpu.VMEM((ROWS,), jnp.int32),
        pltpu.VMEM((ROWS,), jnp.int32),
        pltpu.VMEM((ROWS,), jnp.float32),
        pltpu.VMEM((ROWS, 128), jnp.float32),
        pltpu.VMEM((ROWS, 128), jnp.float32),
        pltpu.VMEM((OW, C), jnp.float32),
        pltpu.SemaphoreType.DMA,
        pltpu.SemaphoreType.DMA,
    ],
)(_roialign_body)


def kernel(inputs, boxes, box_indices):
    imgp = jnp.pad(inputs, ((0, 0), (0, 0), (0, 0), (0, CP - C)))
    img = imgp.reshape(8 * H * W * 2, 128)
    boxes_p = jnp.concatenate(
        [boxes, jnp.zeros((NP - N, 4), boxes.dtype)], axis=0).reshape(NP * 4)
    bidx_p = jnp.concatenate(
        [box_indices, jnp.zeros((NP - N,), box_indices.dtype)])
    return _roialign_sc(img, boxes_p, bidx_p)


# speedup vs baseline: 1.1469x; 1.1469x over previous
import functools

import jax
import jax.numpy as jnp
from jax import lax
from jax.experimental import pallas as pl
from jax.experimental.pallas import tpu as pltpu
from jax.experimental.pallas import tpu_sc as plsc

H = 224
W = 224
C = 192
CP = 256
NPIX = 8 * H * W
N = 1000
NP = 1024
NW = 32
BPW = NP // NW
OH = 7
OW = 7
NCH = C // 16
ROWS = OW * 16

PAD_BLK = 2048


def _pad_body(x_ref, o_ref):
    o_ref[:, pl.ds(0, C)] = x_ref[...]


_pad_rows = pl.pallas_call(
    _pad_body,
    grid=(NPIX // PAD_BLK,),
    in_specs=[pl.BlockSpec((PAD_BLK, C), lambda i: (i, 0))],
    out_specs=pl.BlockSpec((PAD_BLK, CP), lambda i: (i, 0)),
    out_shape=jax.ShapeDtypeStruct((NPIX, CP), jnp.float32),
)


def _axis_params(v):
    t = v.astype(jnp.int32)
    fl = jnp.where(t.astype(jnp.float32) > v, t - 1, t)
    lerp = v - fl.astype(jnp.float32)
    valid = jnp.where((v >= 0.0) & (v <= 223.0), 1.0, 0.0).astype(jnp.float32)
    lo = jnp.clip(fl, 0, 223)
    hi = jnp.clip(fl + 1, 0, 223)
    wlo = valid * (1.0 - lerp)
    whi = valid * lerp
    return lo, hi, wlo, whi


def _roialign_body(img, boxes, bidx, out,
                   boxsm, bism, idxb, wbuf, rows, outrow, sem):
    c = lax.axis_index("c")
    s = lax.axis_index("s")
    wid = s * 2 + c
    lo = wid * BPW
    nb = jnp.minimum(BPW, jnp.maximum(0, N - lo))
    pltpu.sync_copy(boxes.at[pl.ds(lo * 4, BPW * 4)], boxsm.at[pl.ds(0, BPW * 4)])
    pltpu.sync_copy(bidx.at[pl.ds(lo, BPW)], bism.at[pl.ds(0, BPW)])

    lane = lax.iota(jnp.int32, 16)
    sy_hi = ((lane >> 3) & 1) == 1
    sx_hi = ((lane >> 2) & 1) == 1
    cy_hi = ((lane >> 1) & 1) == 1
    cx_hi = (lane & 1) == 1

    def box_loop(j, carry):
        bvec = boxsm[pl.ds(j * 4, 16)]
        bcy = bvec[0]
        bcx = bvec[1]
        bh = bvec[2]
        bw = bvec[3]
        y1 = bcy - bh * 0.5
        y2 = bcy + bh * 0.5
        x1 = bcx - bw * 0.5
        x2 = bcx + bw * 0.5
        bin_h = (y2 - y1) * (1.0 / 7.0)
        bin_w = (x2 - x1) * (1.0 / 7.0)
        gy1 = y1 + 0.25 * bin_h
        gy2 = y2 - 0.25 * bin_h
        gx1 = x1 + 0.25 * bin_w
        gx2 = x2 - 0.25 * bin_w
        hs = (gy2 - gy1) * ((H - 1.0) / 13.0)
        ws = (gx2 - gx1) * ((W - 1.0) / 13.0)
        y0f = gy1 * (H - 1.0)
        x0f = gx1 * (W - 1.0)
        base = bism[pl.ds(j, 16)][0] * (H * W)

        xpar = [_axis_params(x0f + float(ix) * ws) for ix in range(2 * OW)]

        def oy_loop(oy, carry2):
            oyf = (2 * oy).astype(jnp.float32)
            t0, b0, wt0, wb0 = _axis_params(y0f + oyf * hs)
            t1, b1, wt1, wb1 = _axis_params(y0f + (oyf + 1.0) * hs)
            y16 = jnp.where(cy_hi,
                            jnp.where(sy_hi, b1, b0),
                            jnp.where(sy_hi, t1, t0))
            wy16 = 0.25 * jnp.where(cy_hi,
                                    jnp.where(sy_hi, wb1, wb0),
                                    jnp.where(sy_hi, wt1, wt0))
            for ox in range(OW):
                l0, r0, wl0, wr0 = xpar[2 * ox]
                l1, r1, wl1, wr1 = xpar[2 * ox + 1]
                x16 = jnp.where(cx_hi,
                                jnp.where(sx_hi, r1, r0),
                                jnp.where(sx_hi, l1, l0))
                wx16 = jnp.where(cx_hi,
                                 jnp.where(sx_hi, wr1, wr0),
                                 jnp.where(sx_hi, wl1, wl0))
                idxb[pl.ds(ox * 16, 16)] = base + y16 * W + x16
                wbuf[pl.ds(ox * 16, 16)] = wy16 * wx16
            pltpu.async_copy(img.at[idxb], rows, sem).wait()

            def bin_loop(ox, carry3):
                rbase = ox * 16
                wvec = wbuf[pl.ds(rbase, 16)]
                accs = [jnp.zeros((16,), jnp.float32) for _ in range(NCH)]
                for k in range(16):
                    wk = wvec[k]
                    for ch in range(NCH):
                        accs[ch] = (accs[ch]
                                    + rows[rbase + k, pl.ds(ch * 16, 16)] * wk)
                for ch in range(NCH):
                    outrow[ox, pl.ds(ch * 16, 16)] = accs[ch]
                return carry3

            lax.fori_loop(0, OW, bin_loop, None)
            pltpu.sync_copy(outrow, out.at[lo + j, oy])
            return carry2

        lax.fori_loop(0, OH, oy_loop, None)
        return carry

    lax.fori_loop(0, nb, box_loop, None)


_roialign_sc = functools.partial(
    pl.kernel,
    out_type=jax.ShapeDtypeStruct((N, OH, OW, C), jnp.float32),
    mesh=plsc.VectorSubcoreMesh(core_axis_name="c", subcore_axis_name="s"),
    scratch_types=[
        pltpu.VMEM((BPW * 4 + 16,), jnp.float32),
        pltpu.VMEM((BPW + 16,), jnp.int32),
        pltpu.VMEM((ROWS,), jnp.int32),
        pltpu.VMEM((ROWS,), jnp.float32),
        pltpu.VMEM((ROWS, CP), jnp.float32),
        pltpu.VMEM((OW, C), jnp.float32),
        pltpu.SemaphoreType.DMA,
    ],
)(_roialign_body)


def kernel(inputs, boxes, box_indices):
    img = _pad_rows(inputs.reshape(NPIX, C))
    boxes_p = jnp.concatenate(
        [boxes, jnp.zeros((NP - N, 4), boxes.dtype)], axis=0).reshape(NP * 4)
    bidx_p = jnp.concatenate(
        [box_indices, jnp.zeros((NP - N,), box_indices.dtype)])
    return _roialign_sc(img, boxes_p, bidx_p)


# speedup vs baseline: 2.8697x; 2.5022x over previous
import functools

import jax
import jax.numpy as jnp
from jax import lax
from jax.experimental import pallas as pl
from jax.experimental.pallas import tpu as pltpu
from jax.experimental.pallas import tpu_sc as plsc

H = 224
W = 224
C = 192
CP = 256
NPIX = 8 * H * W
N = 1000
NP = 1024
NW = 32
BPW = NP // NW
OH = 7
OW = 7
NCH = C // 16
ROWS = OW * 16

TBLK = 8


def _stage_body(x_ref, o_ref):
    for h in range(TBLK):
        o_ref[pl.ds(h * W, W), pl.ds(0, C)] = jnp.transpose(x_ref[0, h], (1, 0))


_stage_rows = pl.pallas_call(
    _stage_body,
    grid=(8, H // TBLK),
    in_specs=[pl.BlockSpec((1, TBLK, C, W), lambda b, hb: (b, hb, 0, 0))],
    out_specs=pl.BlockSpec(
        (TBLK * W, CP), lambda b, hb: (b * (H // TBLK) + hb, 0)),
    out_shape=jax.ShapeDtypeStruct((NPIX, CP), jnp.float32),
)


def _axis_params(v):
    t = v.astype(jnp.int32)
    fl = jnp.where(t.astype(jnp.float32) > v, t - 1, t)
    lerp = v - fl.astype(jnp.float32)
    valid = jnp.where((v >= 0.0) & (v <= 223.0), 1.0, 0.0).astype(jnp.float32)
    lo = jnp.clip(fl, 0, 223)
    hi = jnp.clip(fl + 1, 0, 223)
    wlo = valid * (1.0 - lerp)
    whi = valid * lerp
    return lo, hi, wlo, whi


def _roialign_body(img, boxes, bidx, out,
                   boxsm, bism, idx0, idx1, w0, w1, rows0, rows1, outrow,
                   sem0, sem1):
    c = lax.axis_index("c")
    s = lax.axis_index("s")
    wid = s * 2 + c
    lo = wid * BPW
    nb = jnp.minimum(BPW, jnp.maximum(0, N - lo))
    pltpu.sync_copy(boxes.at[pl.ds(lo * 4, BPW * 4)], boxsm.at[pl.ds(0, BPW * 4)])
    pltpu.sync_copy(bidx.at[pl.ds(lo, BPW)], bism.at[pl.ds(0, BPW)])

    idxs = (idx0, idx1)
    wbufs = (w0, w1)
    rowss = (rows0, rows1)
    sems = (sem0, sem1)

    lane = lax.iota(jnp.int32, 16)
    sy_hi = ((lane >> 3) & 1) == 1
    sx_hi = ((lane >> 2) & 1) == 1
    cy_hi = ((lane >> 1) & 1) == 1
    cx_hi = (lane & 1) == 1

    def box_loop(j, carry):
        bvec = boxsm[pl.ds(j * 4, 16)]
        bcy = bvec[0]
        bcx = bvec[1]
        bh = bvec[2]
        bw = bvec[3]
        y1 = bcy - bh * 0.5
        y2 = bcy + bh * 0.5
        x1 = bcx - bw * 0.5
        x2 = bcx + bw * 0.5
        bin_h = (y2 - y1) * (1.0 / 7.0)
        bin_w = (x2 - x1) * (1.0 / 7.0)
        gy1 = y1 + 0.25 * bin_h
        gy2 = y2 - 0.25 * bin_h
        gx1 = x1 + 0.25 * bin_w
        gx2 = x2 - 0.25 * bin_w
        hs = (gy2 - gy1) * ((H - 1.0) / 13.0)
        ws = (gx2 - gx1) * ((W - 1.0) / 13.0)
        y0f = gy1 * (H - 1.0)
        x0f = gx1 * (W - 1.0)
        base = bism[pl.ds(j, 16)][0] * (H * W)

        xpar = [_axis_params(x0f + float(ix) * ws) for ix in range(2 * OW)]

        def build(oy, slot):
            t0, b0, wt0, wb0 = _axis_params(y0f + (2.0 * oy) * hs)
            t1, b1, wt1, wb1 = _axis_params(y0f + (2.0 * oy + 1.0) * hs)
            y16 = jnp.where(cy_hi,
                            jnp.where(sy_hi, b1, b0),
                            jnp.where(sy_hi, t1, t0))
            wy16 = 0.25 * jnp.where(cy_hi,
                                    jnp.where(sy_hi, wb1, wb0),
                                    jnp.where(sy_hi, wt1, wt0))
            for ox in range(OW):
                l0, r0, wl0, wr0 = xpar[2 * ox]
                l1, r1, wl1, wr1 = xpar[2 * ox + 1]
                x16 = jnp.where(cx_hi,
                                jnp.where(sx_hi, r1, r0),
                                jnp.where(sx_hi, l1, l0))
                wx16 = jnp.where(cx_hi,
                                 jnp.where(sx_hi, wr1, wr0),
                                 jnp.where(sx_hi, wl1, wl0))
                idxs[slot][pl.ds(ox * 16, 16)] = base + y16 * W + x16
                wbufs[slot][pl.ds(ox * 16, 16)] = wy16 * wx16
            return pltpu.async_copy(img.at[idxs[slot]], rowss[slot], sems[slot])

        cp = build(0, 0)
        for oy in range(OH):
            slot = oy % 2
            nxt = build(oy + 1, 1 - slot) if oy < OH - 1 else None
            cp.wait()
            rows = rowss[slot]
            wbuf = wbufs[slot]

            def bin_loop(ox, carry3):
                rbase = ox * 16
                wvec = wbuf[pl.ds(rbase, 16)]
                accs = [jnp.zeros((16,), jnp.float32) for _ in range(NCH)]
                for k in range(16):
                    wk = wvec[k]
                    for ch in range(NCH):
                        accs[ch] = (accs[ch]
                                    + rows[rbase + k, pl.ds(ch * 16, 16)] * wk)
                for ch in range(NCH):
                    outrow[ox, pl.ds(ch * 16, 16)] = accs[ch]
                return carry3

            lax.fori_loop(0, OW, bin_loop, None)
            pltpu.sync_copy(outrow, out.at[lo + j, oy])
            cp = nxt
        return carry

    lax.fori_loop(0, nb, box_loop, None)


_roialign_sc = functools.partial(
    pl.kernel,
    out_type=jax.ShapeDtypeStruct((N, OH, OW, C), jnp.float32),
    mesh=plsc.VectorSubcoreMesh(core_axis_name="c", subcore_axis_name="s"),
    scratch_types=[
        pltpu.VMEM((BPW * 4 + 16,), jnp.float32),
        pltpu.VMEM((BPW + 16,), jnp.int32),
        pltpu.VMEM((ROWS,), jnp.int32),
        pltpu.VMEM((ROWS,), jnp.int32),
        pltpu.VMEM((ROWS,), jnp.float32),
        pltpu.VMEM((ROWS,), jnp.float32),
        pltpu.VMEM((ROWS, CP), jnp.float32),
        pltpu.VMEM((ROWS, CP), jnp.float32),
        pltpu.VMEM((OW, C), jnp.float32),
        pltpu.SemaphoreType.DMA,
        pltpu.SemaphoreType.DMA,
    ],
)(_roialign_body)


def kernel(inputs, boxes, box_indices):
    img = _stage_rows(jnp.transpose(inputs, (0, 1, 3, 2)))
    boxes_p = jnp.concatenate(
        [boxes, jnp.zeros((NP - N, 4), boxes.dtype)], axis=0).reshape(NP * 4)
    bidx_p = jnp.concatenate(
        [box_indices, jnp.zeros((NP - N,), box_indices.dtype)])
    return _roialign_sc(img, boxes_p, bidx_p)


# speedup vs baseline: 2.9604x; 1.0316x over previous
import functools

import jax
import jax.numpy as jnp
from jax import lax
from jax.experimental import pallas as pl
from jax.experimental.pallas import tpu as pltpu
from jax.experimental.pallas import tpu_sc as plsc

H = 224
W = 224
C = 192
CP = 256
NPIX = 8 * H * W
N = 1000
NP = 1024
NW = 32
BPW = NP // NW
OH = 7
OW = 7
NCH = C // 16
ROWS = OW * 16

TBLK = 8


def _stage_body(x_ref, o_ref):
    for h in range(TBLK):
        o_ref[pl.ds(h * W, W), pl.ds(0, C)] = jnp.transpose(x_ref[0, h], (1, 0))


_stage_rows = pl.pallas_call(
    _stage_body,
    grid=(8, H // TBLK),
    in_specs=[pl.BlockSpec((1, TBLK, C, W), lambda b, hb: (b, hb, 0, 0))],
    out_specs=pl.BlockSpec(
        (TBLK * W, CP), lambda b, hb: (b * (H // TBLK) + hb, 0)),
    out_shape=jax.ShapeDtypeStruct((NPIX, CP), jnp.float32),
)


def _axis_params(v):
    t = v.astype(jnp.int32)
    fl = jnp.where(t.astype(jnp.float32) > v, t - 1, t)
    lerp = v - fl.astype(jnp.float32)
    valid = jnp.where((v >= 0.0) & (v <= 223.0), 1.0, 0.0).astype(jnp.float32)
    lo = jnp.clip(fl, 0, 223)
    hi = jnp.clip(fl + 1, 0, 223)
    wlo = valid * (1.0 - lerp)
    whi = valid * lerp
    return lo, hi, wlo, whi


def _roialign_body(img, boxes, bidx, out,
                   boxsm, bism, idxp, idxa, idxb, wp, wa, wb,
                   rowsp, rowsa, rowsb, outrow, semp, sema, semb):
    c = lax.axis_index("c")
    s = lax.axis_index("s")
    wid = s * 2 + c
    lo = wid * BPW
    nb = jnp.minimum(BPW, jnp.maximum(0, N - lo))
    pltpu.sync_copy(boxes.at[pl.ds(lo * 4, BPW * 4)], boxsm.at[pl.ds(0, BPW * 4)])
    pltpu.sync_copy(bidx.at[pl.ds(lo, BPW)], bism.at[pl.ds(0, BPW)])

    lane = lax.iota(jnp.int32, 16)
    sy_hi = ((lane >> 3) & 1) == 1
    sx_hi = ((lane >> 2) & 1) == 1
    cy_hi = ((lane >> 1) & 1) == 1
    cx_hi = (lane & 1) == 1

    idxs = {"P": idxp, "A": idxa, "B": idxb}
    wbufs = {"P": wp, "A": wa, "B": wb}
    rowss = {"P": rowsp, "A": rowsa, "B": rowsb}
    sems = {"P": semp, "A": sema, "B": semb}

    def slot_of(oy):
        return "P" if oy == 0 else ("A" if oy % 2 == 1 else "B")

    def box_params(j):
        bvec = boxsm[pl.ds(j * 4, 16)]
        bcy = bvec[0]
        bcx = bvec[1]
        bh = bvec[2]
        bw = bvec[3]
        y1 = bcy - bh * 0.5
        y2 = bcy + bh * 0.5
        x1 = bcx - bw * 0.5
        x2 = bcx + bw * 0.5
        bin_h = (y2 - y1) * (1.0 / 7.0)
        bin_w = (x2 - x1) * (1.0 / 7.0)
        gy1 = y1 + 0.25 * bin_h
        gy2 = y2 - 0.25 * bin_h
        gx1 = x1 + 0.25 * bin_w
        gx2 = x2 - 0.25 * bin_w
        hs = (gy2 - gy1) * ((H - 1.0) / 13.0)
        ws = (gx2 - gx1) * ((W - 1.0) / 13.0)
        y0f = gy1 * (H - 1.0)
        x0f = gx1 * (W - 1.0)
        base = jnp.clip(bism[pl.ds(j, 16)][0], 0, 7) * (H * W)
        xpar = [_axis_params(x0f + float(ix) * ws) for ix in range(2 * OW)]
        return y0f, hs, base, xpar

    def build(params, oy, slot):
        y0f, hs, base, xpar = params
        t0, b0, wt0, wb0 = _axis_params(y0f + (2.0 * oy) * hs)
        t1, b1, wt1, wb1 = _axis_params(y0f + (2.0 * oy + 1.0) * hs)
        y16 = jnp.where(cy_hi,
                        jnp.where(sy_hi, b1, b0),
                        jnp.where(sy_hi, t1, t0))
        wy16 = 0.25 * jnp.where(cy_hi,
                                jnp.where(sy_hi, wb1, wb0),
                                jnp.where(sy_hi, wt1, wt0))
        for ox in range(OW):
            l0, r0, wl0, wr0 = xpar[2 * ox]
            l1, r1, wl1, wr1 = xpar[2 * ox + 1]
            x16 = jnp.where(cx_hi,
                            jnp.where(sx_hi, r1, r0),
                            jnp.where(sx_hi, l1, l0))
            wx16 = jnp.where(cx_hi,
                             jnp.where(sx_hi, wr1, wr0),
                             jnp.where(sx_hi, wl1, wl0))
            idxs[slot][pl.ds(ox * 16, 16)] = base + y16 * W + x16
            wbufs[slot][pl.ds(ox * 16, 16)] = wy16 * wx16
        return pltpu.async_copy(img.at[idxs[slot]], rowss[slot], sems[slot])

    def drain_p():
        pltpu.make_async_copy(img.at[idxp], rowsp, semp).wait()

    build(box_params(0), 0, "P")

    def box_loop(j, carry):
        drain_p()
        params = box_params(j)
        cp = None
        for oy in range(OH):
            slot = slot_of(oy)
            if oy < OH - 1:
                nxt = build(params, oy + 1, slot_of(oy + 1))
            else:
                nxt = None
                build(box_params(j + 1), 0, "P")
            if cp is not None:
                cp.wait()
            rows = rowss[slot]
            wbuf = wbufs[slot]

            def bin_loop(ox, carry3):
                rbase = ox * 16
                wvec = wbuf[pl.ds(rbase, 16)]
                accs = [jnp.zeros((16,), jnp.float32) for _ in range(NCH)]
                for k in range(16):
                    wk = wvec[k]
                    for ch in range(NCH):
                        accs[ch] = (accs[ch]
                                    + rows[rbase + k, pl.ds(ch * 16, 16)] * wk)
                for ch in range(NCH):
                    outrow[ox, pl.ds(ch * 16, 16)] = accs[ch]
                return carry3

            lax.fori_loop(0, OW, bin_loop, None)
            pltpu.sync_copy(outrow, out.at[lo + j, oy])
            cp = nxt
        return carry

    lax.fori_loop(0, nb, box_loop, None)
    drain_p()


_roialign_sc = functools.partial(
    pl.kernel,
    out_type=jax.ShapeDtypeStruct((N, OH, OW, C), jnp.float32),
    mesh=plsc.VectorSubcoreMesh(core_axis_name="c", subcore_axis_name="s"),
    scratch_types=[
        pltpu.VMEM((BPW * 4 + 16,), jnp.float32),
        pltpu.VMEM((BPW + 16,), jnp.int32),
        pltpu.VMEM((ROWS,), jnp.int32),
        pltpu.VMEM((ROWS,), jnp.int32),
        pltpu.VMEM((ROWS,), jnp.int32),
        pltpu.VMEM((ROWS,), jnp.float32),
        pltpu.VMEM((ROWS,), jnp.float32),
        pltpu.VMEM((ROWS,), jnp.float32),
        pltpu.VMEM((ROWS, CP), jnp.float32),
        pltpu.VMEM((ROWS, CP), jnp.float32),
        pltpu.VMEM((ROWS, CP), jnp.float32),
        pltpu.VMEM((OW, C), jnp.float32),
        pltpu.SemaphoreType.DMA,
        pltpu.SemaphoreType.DMA,
        pltpu.SemaphoreType.DMA,
    ],
)(_roialign_body)


def kernel(inputs, boxes, box_indices):
    img = _stage_rows(jnp.transpose(inputs, (0, 1, 3, 2)))
    boxes_p = jnp.concatenate(
        [boxes, jnp.zeros((NP - N, 4), boxes.dtype)], axis=0).reshape(NP * 4)
    bidx_p = jnp.concatenate(
        [box_indices, jnp.zeros((NP - N,), box_indices.dtype)])
    return _roialign_sc(img, boxes_p, bidx_p)
